# Initial kernel scaffold; baseline (speedup 1.0000x reference)
import jax
import jax.numpy as jnp
from jax.experimental import pallas as pl


def kernel(image, boxes, labels, anchors):
    raise NotImplementedError("write your pallas kernel here")


# speedup vs baseline: 2.5044x; 2.5044x over previous
import functools

import jax
import jax.numpy as jnp
from jax.experimental import pallas as pl
from jax.experimental.pallas import tpu as pltpu

G = 200
Gp = 256
A = 20000
BA = 512
NB = 40
Ap = BA * NB
IOU_THRESHOLD = 0.5
BIG = 2**30


def _stage1_body(boxes_ref, anch_ref, iou_out, idx_out, gbest_out,
                 acc_iou, acc_idx):
    j = pl.program_id(0)
    bl = boxes_ref[:, 0:1]
    bt = boxes_ref[:, 1:2]
    br = boxes_ref[:, 2:3]
    bb = boxes_ref[:, 3:4]
    al = anch_ref[0:1, :]
    at = anch_ref[1:2, :]
    ar = anch_ref[2:3, :]
    ab = anch_ref[3:4, :]

    w = jnp.maximum(jnp.minimum(br, ar) - jnp.maximum(bl, al), 0.0)
    h = jnp.maximum(jnp.minimum(bb, ab) - jnp.maximum(bt, at), 0.0)
    inter = w * h
    a1 = (br - bl) * (bb - bt)
    a2 = (ar - al) * (ab - at)
    iou = inter / (a1 + a2 - inter)

    gi = jax.lax.broadcasted_iota(jnp.int32, (Gp, BA), 0)
    ai = jax.lax.broadcasted_iota(jnp.int32, (Gp, BA), 1) + j * BA

    m = jnp.max(iou, axis=0, keepdims=True)
    amin = jnp.min(jnp.where(iou == m, gi, BIG), axis=0, keepdims=True)
    iou_out[0:1, :] = m
    idx_out[0:1, :] = amin

    rmax = jnp.max(iou, axis=1, keepdims=True)
    ridx = jnp.min(jnp.where(iou == rmax, ai, BIG), axis=1, keepdims=True)

    @pl.when(j == 0)
    def _():
        acc_iou[:, 0:1] = jnp.full((Gp, 1), -1.0, jnp.float32)

    prev_i = acc_iou[:, 0:1]
    upd = rmax > prev_i
    acc_iou[:, 0:1] = jnp.where(upd, rmax, prev_i)
    @pl.when(j == 0)
    def _():
        acc_idx[:, 0:1] = ridx
    @pl.when(j > 0)
    def _():
        acc_idx[:, 0:1] = jnp.where(upd, ridx, acc_idx[:, 0:1])

    @pl.when(j == NB - 1)
    def _():
        row = jax.lax.broadcasted_iota(jnp.int32, (Gp, 1), 0)
        gbest_out[:, :] = jnp.where(row < G, acc_idx[:, 0:1], -1)


def _stage2_body(iou_ref, idx_ref, gbest_ref, table_ref, anch_ref,
                 bbox_out, lab_out):
    j = pl.program_id(0)
    ai = jax.lax.broadcasted_iota(jnp.int32, (Gp, BA), 1) + j * BA
    gi = jax.lax.broadcasted_iota(jnp.int32, (Gp, BA), 0)

    gb = gbest_ref[:, 0:1]
    eqf = gb == ai
    forced_g = jnp.max(jnp.where(eqf, gi, -1), axis=0, keepdims=True)
    forced = forced_g >= 0
    final_g = jnp.where(forced, forced_g, idx_ref[0:1, :])
    mask = forced | (iou_ref[0:1, :] > IOU_THRESHOLD)

    onehot = (gi == final_g).astype(jnp.float32)
    gath = jax.lax.dot_general(
        table_ref[:, :], onehot, (((1,), (0,)), ((), ())),
        preferred_element_type=jnp.float32,
        precision=jax.lax.Precision.HIGHEST)

    al = anch_ref[0:1, :]
    at = anch_ref[1:2, :]
    ar = anch_ref[2:3, :]
    ab = anch_ref[3:4, :]
    L = jnp.where(mask, gath[0:1, :], al)
    T = jnp.where(mask, gath[1:2, :], at)
    R = jnp.where(mask, gath[2:3, :], ar)
    B = jnp.where(mask, gath[3:4, :], ab)
    bbox_out[0:1, :] = 0.5 * (L + R)
    bbox_out[1:2, :] = 0.5 * (T + B)
    bbox_out[2:3, :] = R - L
    bbox_out[3:4, :] = B - T
    lab = jnp.floor(gath[4:5, :] + 0.5).astype(jnp.int32)
    lab_out[0:1, :] = jnp.where(mask, lab, 0)


@jax.jit
def _run(image, boxes, labels, anchors):
    f32 = jnp.float32
    boxes = boxes.astype(f32)
    anchors = anchors.astype(f32)
    boxes_p = jnp.zeros((Gp, 4), f32).at[:G].set(boxes)
    anch_t = jnp.zeros((4, Ap), f32).at[:, :A].set(anchors.T)
    table_t = (jnp.zeros((8, Gp), f32)
               .at[0:4, :G].set(boxes.T)
               .at[4, :G].set(labels.astype(f32)))

    iou_b, idx_b, gbest = pl.pallas_call(
        _stage1_body,
        grid=(NB,),
        in_specs=[
            pl.BlockSpec((Gp, 4), lambda j: (0, 0)),
            pl.BlockSpec((4, BA), lambda j: (0, j)),
        ],
        out_specs=[
            pl.BlockSpec((1, BA), lambda j: (0, j)),
            pl.BlockSpec((1, BA), lambda j: (0, j)),
            pl.BlockSpec((Gp, 1), lambda j: (0, 0)),
        ],
        out_shape=[
            jax.ShapeDtypeStruct((1, Ap), f32),
            jax.ShapeDtypeStruct((1, Ap), jnp.int32),
            jax.ShapeDtypeStruct((Gp, 1), jnp.int32),
        ],
        scratch_shapes=[
            pltpu.VMEM((Gp, 128), f32),
            pltpu.VMEM((Gp, 128), jnp.int32),
        ],
    )(boxes_p, anch_t)

    bbox_t, lab = pl.pallas_call(
        _stage2_body,
        grid=(NB,),
        in_specs=[
            pl.BlockSpec((1, BA), lambda j: (0, j)),
            pl.BlockSpec((1, BA), lambda j: (0, j)),
            pl.BlockSpec((Gp, 1), lambda j: (0, 0)),
            pl.BlockSpec((8, Gp), lambda j: (0, 0)),
            pl.BlockSpec((4, BA), lambda j: (0, j)),
        ],
        out_specs=[
            pl.BlockSpec((4, BA), lambda j: (0, j)),
            pl.BlockSpec((1, BA), lambda j: (0, j)),
        ],
        out_shape=[
            jax.ShapeDtypeStruct((4, Ap), f32),
            jax.ShapeDtypeStruct((1, Ap), jnp.int32),
        ],
    )(iou_b, idx_b, gbest, table_t, anch_t)

    bboxes_out = bbox_t[:, :A].T
    labels_out = lab[0, :A]
    return (image, bboxes_out, labels_out)


def kernel(image, boxes, labels, anchors):
    return _run(image, boxes, labels, anchors)
